# baseline (speedup 1.0000x reference)
import functools

import jax
import jax.numpy as jnp
from jax import lax
from jax.experimental import pallas as pl
from jax.experimental.pallas import tpu as pltpu
from jax.experimental.pallas import tpu_sc as plsc

BATCH = 16384
EMB_K = 16
NROWS = 1000000

_NC = 2
_NS = 16
_L = 16
_NW = _NC * _NS
_BPW = BATCH // _NW
_CHUNK = 128
_NCHUNK = _BPW // _CHUNK
_NBLK = _BPW // _L

_TB = 8192
_TSTEPS = -(-NROWS // _TB)


def _transpose_body(wt_ref, ht_ref, wout_ref, hout_ref):
    eye = (lax.broadcasted_iota(jnp.int32, (EMB_K, EMB_K), 0)
           == lax.broadcasted_iota(jnp.int32, (EMB_K, EMB_K), 1)
           ).astype(jnp.float32)
    wout_ref[...] = lax.dot_general(
        wt_ref[...], eye, (((0,), (0,)), ((), ())),
        precision=lax.Precision.HIGHEST,
        preferred_element_type=jnp.float32)
    hout_ref[...] = lax.dot_general(
        ht_ref[...], eye, (((0,), (0,)), ((), ())),
        precision=lax.Precision.HIGHEST,
        preferred_element_type=jnp.float32)


@functools.cache
def _make_transpose_tc():
    return pl.pallas_call(
        _transpose_body,
        grid=(_TSTEPS,),
        in_specs=[pl.BlockSpec((EMB_K, _TB), lambda i: (0, i)),
                  pl.BlockSpec((EMB_K, _TB), lambda i: (0, i))],
        out_specs=[pl.BlockSpec((_TB, EMB_K), lambda i: (i, 0)),
                   pl.BlockSpec((_TB, EMB_K), lambda i: (i, 0))],
        out_shape=[jax.ShapeDtypeStruct((NROWS, EMB_K), jnp.float32),
                   jax.ShapeDtypeStruct((NROWS, EMB_K), jnp.float32)],
        compiler_params=pltpu.CompilerParams(skip_device_barrier=True),
    )


def _ncf_body(uidx_hbm, vidx_hbm, w_hbm, h_hbm, w1_hbm, b1_hbm, w2_hbm,
              out_hbm,
              uidx_v, vidx_v, urows_v, vrows_v,
              w1_v, b1_v, w2_v, out_v,
              sem_u, sem_v):
    wid = lax.axis_index("s") * _NC + lax.axis_index("c")
    crow = wid * _NCHUNK
    pltpu.sync_copy(uidx_hbm.at[pl.ds(crow, _NCHUNK)], uidx_v)
    pltpu.sync_copy(vidx_hbm.at[pl.ds(crow, _NCHUNK)], vidx_v)

    copies = []
    for c in range(_NCHUNK):
        copies.append(pltpu.async_copy(
            w_hbm.at[uidx_v.at[c]],
            urows_v.at[pl.ds(c * _CHUNK, _CHUNK)], sem_u))
        copies.append(pltpu.async_copy(
            h_hbm.at[vidx_v.at[c]],
            vrows_v.at[pl.ds(c * _CHUNK, _CHUNK)], sem_v))
    pltpu.sync_copy(w1_hbm, w1_v)
    pltpu.sync_copy(b1_hbm, b1_v)
    pltpu.sync_copy(w2_hbm, w2_v)
    for cp in copies:
        cp.wait()

    lane = lax.iota(jnp.int32, _L)
    b1_vec = b1_v[...]
    w2_vec = w2_v[...]

    def block(blk, carry):
        row_ids = blk * _L + lane
        ucols = [plsc.load_gather(urows_v,
                                  [row_ids, jnp.full((_L,), k, jnp.int32)])
                 for k in range(EMB_K)]
        vcols = [plsc.load_gather(vrows_v,
                                  [row_ids, jnp.full((_L,), k, jnp.int32)])
                 for k in range(EMB_K)]
        acc = jnp.zeros((_L,), jnp.float32)
        for j in range(EMB_K):
            w1u = w1_v[j, pl.ds(0, EMB_K)]
            w1v = w1_v[j, pl.ds(EMB_K, EMB_K)]
            h = jnp.full((_L,), b1_vec[j], jnp.float32)
            for k in range(EMB_K):
                h = h + ucols[k] * w1u[k]
            for k in range(EMB_K):
                h = h + vcols[k] * w1v[k]
            h = jnp.maximum(h, 0.0)
            acc = acc + h * w2_vec[j]
        out_v[pl.ds(blk * _L, _L)] = acc
        return carry

    lax.fori_loop(0, _NBLK, block, 0)

    pltpu.sync_copy(out_v, out_hbm.at[pl.ds(wid * _BPW, _BPW)])


@functools.cache
def _make_ncf_sc():
  return functools.partial(
    pl.kernel,
    out_type=jax.ShapeDtypeStruct((BATCH,), jnp.float32),
    mesh=plsc.VectorSubcoreMesh(core_axis_name="c", subcore_axis_name="s",
                                num_cores=_NC),
    compiler_params=pltpu.CompilerParams(needs_layout_passes=False,
                                         use_tc_tiling_on_sc=False,
                                         skip_device_barrier=True),
    scratch_types=[
        pltpu.VMEM((_NCHUNK, _CHUNK), jnp.int32),
        pltpu.VMEM((_NCHUNK, _CHUNK), jnp.int32),
        pltpu.VMEM((_BPW, EMB_K), jnp.float32),
        pltpu.VMEM((_BPW, EMB_K), jnp.float32),
        pltpu.VMEM((EMB_K, 2 * EMB_K), jnp.float32),
        pltpu.VMEM((EMB_K,), jnp.float32),
        pltpu.VMEM((EMB_K,), jnp.float32),
        pltpu.VMEM((_BPW,), jnp.float32),
        pltpu.SemaphoreType.DMA,
        pltpu.SemaphoreType.DMA,
    ],
  )(_ncf_body)


def kernel(x, W_table, H_table, W1, b1, W2):
    u_idx = x[:, 0].reshape(BATCH // _CHUNK, _CHUNK)
    v_idx = x[:, 1].reshape(BATCH // _CHUNK, _CHUNK)
    w_lin, h_lin = _make_transpose_tc()(W_table.T, H_table.T)
    out = _make_ncf_sc()(u_idx, v_idx, w_lin, h_lin, W1, b1,
                         W2.reshape(EMB_K))
    return out.reshape(BATCH, 1)


# speedup vs baseline: 6.3100x; 6.3100x over previous
import functools

import jax
import jax.numpy as jnp
from jax import lax
from jax.experimental import pallas as pl
from jax.experimental.pallas import tpu as pltpu
from jax.experimental.pallas import tpu_sc as plsc

BATCH = 16384
EMB_K = 16
NROWS = 1000000

_NC = 2
_NS = 16
_L = 16
_NW = _NC * _NS
_BPW = BATCH // _NW
_CHUNK = 128
_NCHUNK = _BPW // _CHUNK
_NBLK = _BPW // _L

_JB = 64
_BCOLS = _JB * 128
_NJ = -(-NROWS // _BCOLS)
_BRROWS = _NJ * _JB * 8
_GSTRIDE = _BRROWS * 128 // EMB_K
_N16 = 2 * _GSTRIDE


def _detile_body(wt_ref, ht_ref, wb_ref, hb_ref):
    for src, dst in ((wt_ref, wb_ref), (ht_ref, hb_ref)):
        x = src[...]
        x = x.reshape(8, _JB, 128).transpose(1, 0, 2)
        dst[...] = x.reshape(1, _JB * 8, 128)


@functools.cache
def _make_detile_tc():
    return pl.pallas_call(
        _detile_body,
        grid=(2, _NJ),
        in_specs=[pl.BlockSpec((8, _BCOLS), lambda g, j: (g, j)),
                  pl.BlockSpec((8, _BCOLS), lambda g, j: (g, j))],
        out_specs=[pl.BlockSpec((1, _JB * 8, 128), lambda g, j: (g, j, 0)),
                   pl.BlockSpec((1, _JB * 8, 128), lambda g, j: (g, j, 0))],
        out_shape=[jax.ShapeDtypeStruct((2, _BRROWS, 128), jnp.float32),
                   jax.ShapeDtypeStruct((2, _BRROWS, 128), jnp.float32)],
    )


def _ncf_body(uidx_hbm, vidx_hbm, wb_hbm, hb_hbm, w1_hbm, b1_hbm, w2_hbm,
              out_hbm,
              uidx_v, vidx_v, qu_v, qv_v, lu_v, lv_v, m_v,
              gu_v, gv_v, ucolsT, vcolsT,
              w1_v, b1_v, w2_v, out_v,
              sem_u, sem_v):
    wid = lax.axis_index("s") * _NC + lax.axis_index("c")
    crow = wid * _NCHUNK
    pltpu.sync_copy(uidx_hbm.at[pl.ds(crow, _NCHUNK)], uidx_v)
    pltpu.sync_copy(vidx_hbm.at[pl.ds(crow, _NCHUNK)], vidx_v)
    pltpu.sync_copy(w1_hbm, w1_v)
    pltpu.sync_copy(b1_hbm, b1_v)
    pltpu.sync_copy(w2_hbm, w2_v)

    for idx_ref, q_ref, l_ref in ((uidx_v, qu_v, lu_v),
                                  (vidx_v, qv_v, lv_v)):
        for c in range(_NCHUNK):
            for s in range(_CHUNK // _L):
                idx = idx_ref[c, pl.ds(s * _L, _L)]
                q = ((jnp.right_shift(idx, 7) * 64)
                     | (jnp.right_shift(idx, 4) & 7))
                q_ref[pl.ds(c * _CHUNK + s * _L, _L)] = q
                l_ref[pl.ds(c * _CHUNK + s * _L, _L)] = idx & 15

    lane = lax.iota(jnp.int32, _L)
    b1_vec = b1_v[...]
    w2_vec = w2_v[...]

    def extract(gbuf, lanes, colsT, k):
        def body(b, carry):
            row_ids = b * _L + lane
            lvec = lanes[pl.ds(b * _L, _L)]
            val = plsc.load_gather(gbuf, [row_ids, lvec])
            colsT[k, pl.ds(b * _L, _L)] = val
            return carry
        lax.fori_loop(0, _NBLK, body, 0)

    for k in range(EMB_K):
        kconst = (k // 8) * _GSTRIDE + (k % 8) * 8
        for c in range(_NCHUNK):
            for s in range(_CHUNK // _L):
                off = c * _CHUNK + s * _L
                m_v[c, pl.ds(s * _L, _L)] = (
                    qu_v[pl.ds(off, _L)] + kconst)
                m_v[4 + c, pl.ds(s * _L, _L)] = (
                    qv_v[pl.ds(off, _L)] + kconst)
        copies = []
        for c in range(_NCHUNK):
            copies.append(pltpu.async_copy(
                wb_hbm.at[m_v.at[c]],
                gu_v.at[pl.ds(c * _CHUNK, _CHUNK)], sem_u))
            copies.append(pltpu.async_copy(
                hb_hbm.at[m_v.at[4 + c]],
                gv_v.at[pl.ds(c * _CHUNK, _CHUNK)], sem_v))
        for cp in copies:
            cp.wait()
        extract(gu_v, lu_v, ucolsT, k)
        extract(gv_v, lv_v, vcolsT, k)

    def block(blk, carry):
        base = blk * _L
        ucols = [ucolsT[k, pl.ds(base, _L)] for k in range(EMB_K)]
        vcols = [vcolsT[k, pl.ds(base, _L)] for k in range(EMB_K)]
        acc = jnp.zeros((_L,), jnp.float32)
        for j in range(EMB_K):
            w1u = w1_v[j, pl.ds(0, EMB_K)]
            w1v = w1_v[j, pl.ds(EMB_K, EMB_K)]
            h = jnp.full((_L,), b1_vec[j], jnp.float32)
            for k in range(EMB_K):
                h = h + ucols[k] * w1u[k]
            for k in range(EMB_K):
                h = h + vcols[k] * w1v[k]
            h = jnp.maximum(h, 0.0)
            acc = acc + h * w2_vec[j]
        out_v[pl.ds(base, _L)] = acc
        return carry

    lax.fori_loop(0, _NBLK, block, 0)

    pltpu.sync_copy(out_v, out_hbm.at[pl.ds(wid * _BPW, _BPW)])


@functools.cache
def _make_ncf_sc():
  return functools.partial(
    pl.kernel,
    out_type=jax.ShapeDtypeStruct((BATCH,), jnp.float32),
    mesh=plsc.VectorSubcoreMesh(core_axis_name="c", subcore_axis_name="s",
                                num_cores=_NC),
    compiler_params=pltpu.CompilerParams(needs_layout_passes=False,
                                         use_tc_tiling_on_sc=False),
    scratch_types=[
        pltpu.VMEM((_NCHUNK, _CHUNK), jnp.int32),
        pltpu.VMEM((_NCHUNK, _CHUNK), jnp.int32),
        pltpu.VMEM((_BPW,), jnp.int32),
        pltpu.VMEM((_BPW,), jnp.int32),
        pltpu.VMEM((_BPW,), jnp.int32),
        pltpu.VMEM((_BPW,), jnp.int32),
        pltpu.VMEM((2 * _NCHUNK, _CHUNK), jnp.int32),
        pltpu.VMEM((_BPW, EMB_K), jnp.float32),
        pltpu.VMEM((_BPW, EMB_K), jnp.float32),
        pltpu.VMEM((EMB_K, _BPW), jnp.float32),
        pltpu.VMEM((EMB_K, _BPW), jnp.float32),
        pltpu.VMEM((EMB_K, 2 * EMB_K), jnp.float32),
        pltpu.VMEM((EMB_K,), jnp.float32),
        pltpu.VMEM((EMB_K,), jnp.float32),
        pltpu.VMEM((_BPW,), jnp.float32),
        pltpu.SemaphoreType.DMA,
        pltpu.SemaphoreType.DMA,
    ],
  )(_ncf_body)


def kernel(x, W_table, H_table, W1, b1, W2):
    u_idx = x[:, 0].reshape(BATCH // _CHUNK, _CHUNK)
    v_idx = x[:, 1].reshape(BATCH // _CHUNK, _CHUNK)
    wb, hb = _make_detile_tc()(W_table.T, H_table.T)
    out = _make_ncf_sc()(u_idx, v_idx,
                         wb.reshape(_N16, EMB_K), hb.reshape(_N16, EMB_K),
                         W1, b1, W2.reshape(EMB_K))
    return out.reshape(BATCH, 1)


# speedup vs baseline: 7.7004x; 1.2203x over previous
import functools

import jax
import jax.numpy as jnp
from jax import lax
from jax.experimental import pallas as pl
from jax.experimental.pallas import tpu as pltpu
from jax.experimental.pallas import tpu_sc as plsc

BATCH = 16384
EMB_K = 16
NROWS = 1000000

_NC = 2
_NS = 16
_L = 16
_NW = _NC * _NS
_BPW = BATCH // _NW
_CHUNK = 128
_NCHUNK = _BPW // _CHUNK
_NBLK = _BPW // _L

_JB = 128
_BCOLS = _JB * 128
_NJ = -(-NROWS // _BCOLS)
_BRROWS = _NJ * _JB * 8
_GSTRIDE = _BRROWS * 128 // EMB_K
_N16 = 2 * _GSTRIDE


def _detile_body(wt_ref, ht_ref, wb_ref, hb_ref):
    for src, dst in ((wt_ref, wb_ref), (ht_ref, hb_ref)):
        x = src[...]
        x = x.reshape(8, _JB, 128).transpose(1, 0, 2)
        dst[...] = x.reshape(1, _JB * 8, 128)


@functools.cache
def _make_detile_tc():
    return pl.pallas_call(
        _detile_body,
        grid=(2, _NJ),
        in_specs=[pl.BlockSpec((8, _BCOLS), lambda g, j: (g, j)),
                  pl.BlockSpec((8, _BCOLS), lambda g, j: (g, j))],
        out_specs=[pl.BlockSpec((1, _JB * 8, 128), lambda g, j: (g, j, 0)),
                   pl.BlockSpec((1, _JB * 8, 128), lambda g, j: (g, j, 0))],
        out_shape=[jax.ShapeDtypeStruct((2, _BRROWS, 128), jnp.float32),
                   jax.ShapeDtypeStruct((2, _BRROWS, 128), jnp.float32)],
    )


def _ncf_body(uidx_hbm, vidx_hbm, wb_hbm, hb_hbm, w1_hbm, b1_hbm, w2_hbm,
              out_hbm,
              uidx_v, vidx_v, qu_v, qv_v, lu_v, lv_v, m_v,
              gu_v, gv_v, ucolsT, vcolsT,
              w1_v, b1_v, w2_v, out_v,
              sem_u, sem_v):
    wid = lax.axis_index("s") * _NC + lax.axis_index("c")
    crow = wid * _NCHUNK
    pltpu.sync_copy(uidx_hbm.at[pl.ds(crow, _NCHUNK)], uidx_v)
    pltpu.sync_copy(vidx_hbm.at[pl.ds(crow, _NCHUNK)], vidx_v)
    pltpu.sync_copy(w1_hbm, w1_v)
    pltpu.sync_copy(b1_hbm, b1_v)
    pltpu.sync_copy(w2_hbm, w2_v)

    for idx_ref, q_ref, l_ref in ((uidx_v, qu_v, lu_v),
                                  (vidx_v, qv_v, lv_v)):
        for c in range(_NCHUNK):
            for s in range(_CHUNK // _L):
                idx = idx_ref[c, pl.ds(s * _L, _L)]
                q = ((jnp.right_shift(idx, 7) * 64)
                     | (jnp.right_shift(idx, 4) & 7))
                q_ref[pl.ds(c * _CHUNK + s * _L, _L)] = q
                l_ref[pl.ds(c * _CHUNK + s * _L, _L)] = idx & 15

    lane = lax.iota(jnp.int32, _L)
    b1_vec = b1_v[...]
    w2_vec = w2_v[...]

    def extract(gbuf, lanes, colsT, k):
        def body(b, carry):
            row_ids = b * _L + lane
            lvec = lanes[pl.ds(b * _L, _L)]
            val = plsc.load_gather(gbuf, [row_ids, lvec])
            colsT[k, pl.ds(b * _L, _L)] = val
            return carry
        lax.fori_loop(0, _NBLK, body, 0)

    for k in range(EMB_K):
        kconst = (k // 8) * _GSTRIDE + (k % 8) * 8
        for c in range(_NCHUNK):
            for s in range(_CHUNK // _L):
                off = c * _CHUNK + s * _L
                m_v[c, pl.ds(s * _L, _L)] = (
                    qu_v[pl.ds(off, _L)] + kconst)
                m_v[4 + c, pl.ds(s * _L, _L)] = (
                    qv_v[pl.ds(off, _L)] + kconst)
        copies = []
        for c in range(_NCHUNK):
            copies.append(pltpu.async_copy(
                wb_hbm.at[m_v.at[c]],
                gu_v.at[pl.ds(c * _CHUNK, _CHUNK)], sem_u))
            copies.append(pltpu.async_copy(
                hb_hbm.at[m_v.at[4 + c]],
                gv_v.at[pl.ds(c * _CHUNK, _CHUNK)], sem_v))
        for cp in copies:
            cp.wait()
        extract(gu_v, lu_v, ucolsT, k)
        extract(gv_v, lv_v, vcolsT, k)

    def block(blk, carry):
        base = blk * _L
        ucols = [ucolsT[k, pl.ds(base, _L)] for k in range(EMB_K)]
        vcols = [vcolsT[k, pl.ds(base, _L)] for k in range(EMB_K)]
        acc = jnp.zeros((_L,), jnp.float32)
        for j in range(EMB_K):
            w1u = w1_v[j, pl.ds(0, EMB_K)]
            w1v = w1_v[j, pl.ds(EMB_K, EMB_K)]
            h = jnp.full((_L,), b1_vec[j], jnp.float32)
            for k in range(EMB_K):
                h = h + ucols[k] * w1u[k]
            for k in range(EMB_K):
                h = h + vcols[k] * w1v[k]
            h = jnp.maximum(h, 0.0)
            acc = acc + h * w2_vec[j]
        out_v[pl.ds(base, _L)] = acc
        return carry

    lax.fori_loop(0, _NBLK, block, 0)

    pltpu.sync_copy(out_v, out_hbm.at[pl.ds(wid * _BPW, _BPW)])


@functools.cache
def _make_ncf_sc():
  return functools.partial(
    pl.kernel,
    out_type=jax.ShapeDtypeStruct((BATCH,), jnp.float32),
    mesh=plsc.VectorSubcoreMesh(core_axis_name="c", subcore_axis_name="s",
                                num_cores=_NC),
    compiler_params=pltpu.CompilerParams(needs_layout_passes=False,
                                         use_tc_tiling_on_sc=False),
    scratch_types=[
        pltpu.VMEM((_NCHUNK, _CHUNK), jnp.int32),
        pltpu.VMEM((_NCHUNK, _CHUNK), jnp.int32),
        pltpu.VMEM((_BPW,), jnp.int32),
        pltpu.VMEM((_BPW,), jnp.int32),
        pltpu.VMEM((_BPW,), jnp.int32),
        pltpu.VMEM((_BPW,), jnp.int32),
        pltpu.VMEM((2 * _NCHUNK, _CHUNK), jnp.int32),
        pltpu.VMEM((_BPW, EMB_K), jnp.float32),
        pltpu.VMEM((_BPW, EMB_K), jnp.float32),
        pltpu.VMEM((EMB_K, _BPW), jnp.float32),
        pltpu.VMEM((EMB_K, _BPW), jnp.float32),
        pltpu.VMEM((EMB_K, 2 * EMB_K), jnp.float32),
        pltpu.VMEM((EMB_K,), jnp.float32),
        pltpu.VMEM((EMB_K,), jnp.float32),
        pltpu.VMEM((_BPW,), jnp.float32),
        pltpu.SemaphoreType.DMA,
        pltpu.SemaphoreType.DMA,
    ],
  )(_ncf_body)


def kernel(x, W_table, H_table, W1, b1, W2):
    u_idx = x[:, 0].reshape(BATCH // _CHUNK, _CHUNK)
    v_idx = x[:, 1].reshape(BATCH // _CHUNK, _CHUNK)
    wb, hb = _make_detile_tc()(W_table.T, H_table.T)
    out = _make_ncf_sc()(u_idx, v_idx,
                         wb.reshape(_N16, EMB_K), hb.reshape(_N16, EMB_K),
                         W1, b1, W2.reshape(EMB_K))
    return out.reshape(BATCH, 1)


# speedup vs baseline: 9.4946x; 1.2330x over previous
import functools

import jax
import jax.numpy as jnp
from jax import lax
from jax.experimental import pallas as pl
from jax.experimental.pallas import tpu as pltpu
from jax.experimental.pallas import tpu_sc as plsc

BATCH = 16384
EMB_K = 16
NROWS = 1000000

_NC = 2
_NS = 16
_L = 16
_NW = _NC * _NS
_BPW = BATCH // _NW
_CHUNK = 128
_NCHUNK = _BPW // _CHUNK
_NBLK = _BPW // _L

_JB = 256
_BCOLS = _JB * 128
_NJ = -(-NROWS // _BCOLS)
_BRROWS = _NJ * _JB * 8
_GSTRIDE = _BRROWS * 128 // EMB_K
_N16 = 2 * _GSTRIDE


def _detile_body(wt_ref, ht_ref, wb_ref, hb_ref):
    for src, dst in ((wt_ref, wb_ref), (ht_ref, hb_ref)):
        x = src[...]
        x = x.reshape(8, _JB, 128).transpose(1, 0, 2)
        dst[...] = x.reshape(1, _JB * 8, 128)


@functools.cache
def _make_detile_tc():
    return pl.pallas_call(
        _detile_body,
        grid=(2, _NJ),
        in_specs=[pl.BlockSpec((8, _BCOLS), lambda g, j: (g, j)),
                  pl.BlockSpec((8, _BCOLS), lambda g, j: (g, j))],
        out_specs=[pl.BlockSpec((1, _JB * 8, 128), lambda g, j: (g, j, 0)),
                   pl.BlockSpec((1, _JB * 8, 128), lambda g, j: (g, j, 0))],
        out_shape=[jax.ShapeDtypeStruct((2, _BRROWS, 128), jnp.float32),
                   jax.ShapeDtypeStruct((2, _BRROWS, 128), jnp.float32)],
    )


def _ncf_body(uidx_hbm, vidx_hbm, wb_hbm, hb_hbm, w1_hbm, b1_hbm, w2_hbm,
              out_hbm,
              uidx_v, vidx_v, qu_v, qv_v, lu_v, lv_v, m_v,
              gu_v, gv_v, ucolsT, vcolsT,
              w1_v, b1_v, w2_v, out_v,
              sem_u, sem_v):
    wid = lax.axis_index("s") * _NC + lax.axis_index("c")
    crow = wid * _NCHUNK
    pltpu.sync_copy(uidx_hbm.at[pl.ds(crow, _NCHUNK)], uidx_v)
    pltpu.sync_copy(vidx_hbm.at[pl.ds(crow, _NCHUNK)], vidx_v)
    pltpu.sync_copy(w1_hbm, w1_v)
    pltpu.sync_copy(b1_hbm, b1_v)
    pltpu.sync_copy(w2_hbm, w2_v)

    for idx_ref, q_ref, l_ref in ((uidx_v, qu_v, lu_v),
                                  (vidx_v, qv_v, lv_v)):
        for c in range(_NCHUNK):
            for s in range(_CHUNK // _L):
                idx = idx_ref[c, pl.ds(s * _L, _L)]
                q = ((jnp.right_shift(idx, 7) * 64)
                     | (jnp.right_shift(idx, 4) & 7))
                q_ref[pl.ds(c * _CHUNK + s * _L, _L)] = q
                l_ref[pl.ds(c * _CHUNK + s * _L, _L)] = idx & 15

    lane = lax.iota(jnp.int32, _L)
    b1_vec = b1_v[...]
    w2_vec = w2_v[...]

    def extract(gbuf, lanes, colsT, k):
        def body(b, carry):
            row_ids = b * _L + lane
            lvec = lanes[pl.ds(b * _L, _L)]
            val = plsc.load_gather(gbuf, [row_ids, lvec])
            colsT[k, pl.ds(b * _L, _L)] = val
            return carry
        lax.fori_loop(0, _NBLK, body, 0)

    for k in range(EMB_K):
        kconst = (k // 8) * _GSTRIDE + (k % 8) * 8
        for c in range(_NCHUNK):
            for s in range(_CHUNK // _L):
                off = c * _CHUNK + s * _L
                m_v[c, pl.ds(s * _L, _L)] = (
                    qu_v[pl.ds(off, _L)] + kconst)
                m_v[4 + c, pl.ds(s * _L, _L)] = (
                    qv_v[pl.ds(off, _L)] + kconst)
        copies = []
        for c in range(_NCHUNK):
            copies.append(pltpu.async_copy(
                wb_hbm.at[m_v.at[c]],
                gu_v.at[pl.ds(c * _CHUNK, _CHUNK)], sem_u))
            copies.append(pltpu.async_copy(
                hb_hbm.at[m_v.at[4 + c]],
                gv_v.at[pl.ds(c * _CHUNK, _CHUNK)], sem_v))
        for cp in copies:
            cp.wait()
        extract(gu_v, lu_v, ucolsT, k)
        extract(gv_v, lv_v, vcolsT, k)

    def block(blk, carry):
        base = blk * _L
        ucols = [ucolsT[k, pl.ds(base, _L)] for k in range(EMB_K)]
        vcols = [vcolsT[k, pl.ds(base, _L)] for k in range(EMB_K)]
        acc = jnp.zeros((_L,), jnp.float32)
        for j in range(EMB_K):
            w1u = w1_v[j, pl.ds(0, EMB_K)]
            w1v = w1_v[j, pl.ds(EMB_K, EMB_K)]
            h = jnp.full((_L,), b1_vec[j], jnp.float32)
            for k in range(EMB_K):
                h = h + ucols[k] * w1u[k]
            for k in range(EMB_K):
                h = h + vcols[k] * w1v[k]
            h = jnp.maximum(h, 0.0)
            acc = acc + h * w2_vec[j]
        out_v[pl.ds(base, _L)] = acc
        return carry

    lax.fori_loop(0, _NBLK, block, 0)

    pltpu.sync_copy(out_v, out_hbm.at[pl.ds(wid * _BPW, _BPW)])


@functools.cache
def _make_ncf_sc():
  return functools.partial(
    pl.kernel,
    out_type=jax.ShapeDtypeStruct((BATCH,), jnp.float32),
    mesh=plsc.VectorSubcoreMesh(core_axis_name="c", subcore_axis_name="s",
                                num_cores=_NC),
    compiler_params=pltpu.CompilerParams(needs_layout_passes=False,
                                         use_tc_tiling_on_sc=False),
    scratch_types=[
        pltpu.VMEM((_NCHUNK, _CHUNK), jnp.int32),
        pltpu.VMEM((_NCHUNK, _CHUNK), jnp.int32),
        pltpu.VMEM((_BPW,), jnp.int32),
        pltpu.VMEM((_BPW,), jnp.int32),
        pltpu.VMEM((_BPW,), jnp.int32),
        pltpu.VMEM((_BPW,), jnp.int32),
        pltpu.VMEM((2 * _NCHUNK, _CHUNK), jnp.int32),
        pltpu.VMEM((_BPW, EMB_K), jnp.float32),
        pltpu.VMEM((_BPW, EMB_K), jnp.float32),
        pltpu.VMEM((EMB_K, _BPW), jnp.float32),
        pltpu.VMEM((EMB_K, _BPW), jnp.float32),
        pltpu.VMEM((EMB_K, 2 * EMB_K), jnp.float32),
        pltpu.VMEM((EMB_K,), jnp.float32),
        pltpu.VMEM((EMB_K,), jnp.float32),
        pltpu.VMEM((_BPW,), jnp.float32),
        pltpu.SemaphoreType.DMA,
        pltpu.SemaphoreType.DMA,
    ],
  )(_ncf_body)


def kernel(x, W_table, H_table, W1, b1, W2):
    u_idx = x[:, 0].reshape(BATCH // _CHUNK, _CHUNK)
    v_idx = x[:, 1].reshape(BATCH // _CHUNK, _CHUNK)
    wb, hb = _make_detile_tc()(W_table.T, H_table.T)
    out = _make_ncf_sc()(u_idx, v_idx,
                         wb.reshape(_N16, EMB_K), hb.reshape(_N16, EMB_K),
                         W1, b1, W2.reshape(EMB_K))
    return out.reshape(BATCH, 1)


# speedup vs baseline: 10.3248x; 1.0874x over previous
import functools

import jax
import jax.numpy as jnp
from jax import lax
from jax.experimental import pallas as pl
from jax.experimental.pallas import tpu as pltpu
from jax.experimental.pallas import tpu_sc as plsc

BATCH = 16384
EMB_K = 16
NROWS = 1000000

_NC = 2
_NS = 16
_L = 16
_NW = _NC * _NS
_BPW = BATCH // _NW
_CHUNK = 128
_NCHUNK = _BPW // _CHUNK
_NBLK = _BPW // _L

_JB = 512
_BCOLS = _JB * 128
_NJ = -(-NROWS // _BCOLS)
_BRROWS = _NJ * _JB * 8
_GSTRIDE = _BRROWS * 128 // EMB_K
_N16 = 2 * _GSTRIDE


def _detile_body(wt_ref, ht_ref, wb_ref, hb_ref):
    for src, dst in ((wt_ref, wb_ref), (ht_ref, hb_ref)):
        x = src[...]
        x = x.reshape(8, _JB, 128).transpose(1, 0, 2)
        dst[...] = x.reshape(1, _JB * 8, 128)


@functools.cache
def _make_detile_tc():
    return pl.pallas_call(
        _detile_body,
        grid=(2, _NJ),
        in_specs=[pl.BlockSpec((8, _BCOLS), lambda g, j: (g, j)),
                  pl.BlockSpec((8, _BCOLS), lambda g, j: (g, j))],
        out_specs=[pl.BlockSpec((1, _JB * 8, 128), lambda g, j: (g, j, 0)),
                   pl.BlockSpec((1, _JB * 8, 128), lambda g, j: (g, j, 0))],
        out_shape=[jax.ShapeDtypeStruct((2, _BRROWS, 128), jnp.float32),
                   jax.ShapeDtypeStruct((2, _BRROWS, 128), jnp.float32)],
    )


def _ncf_body(uidx_hbm, vidx_hbm, wb_hbm, hb_hbm, w1_hbm, b1_hbm, w2_hbm,
              out_hbm,
              uidx_v, vidx_v, qu_v, qv_v, lu_v, lv_v, m_v,
              gu_v, gv_v, ucolsT, vcolsT,
              w1_v, b1_v, w2_v, out_v,
              sem_u, sem_v):
    wid = lax.axis_index("s") * _NC + lax.axis_index("c")
    crow = wid * _NCHUNK
    pltpu.sync_copy(uidx_hbm.at[pl.ds(crow, _NCHUNK)], uidx_v)
    pltpu.sync_copy(vidx_hbm.at[pl.ds(crow, _NCHUNK)], vidx_v)
    pltpu.sync_copy(w1_hbm, w1_v)
    pltpu.sync_copy(b1_hbm, b1_v)
    pltpu.sync_copy(w2_hbm, w2_v)

    for idx_ref, q_ref, l_ref in ((uidx_v, qu_v, lu_v),
                                  (vidx_v, qv_v, lv_v)):
        for c in range(_NCHUNK):
            for s in range(_CHUNK // _L):
                idx = idx_ref[c, pl.ds(s * _L, _L)]
                q = ((jnp.right_shift(idx, 7) * 64)
                     | (jnp.right_shift(idx, 4) & 7))
                q_ref[pl.ds(c * _CHUNK + s * _L, _L)] = q
                l_ref[pl.ds(c * _CHUNK + s * _L, _L)] = idx & 15

    lane = lax.iota(jnp.int32, _L)
    b1_vec = b1_v[...]
    w2_vec = w2_v[...]

    def extract(gbuf, lanes, colsT, k):
        def body(b, carry):
            row_ids = b * _L + lane
            lvec = lanes[pl.ds(b * _L, _L)]
            val = plsc.load_gather(gbuf, [row_ids, lvec])
            colsT[k, pl.ds(b * _L, _L)] = val
            return carry
        lax.fori_loop(0, _NBLK, body, 0)

    for k in range(EMB_K):
        kconst = (k // 8) * _GSTRIDE + (k % 8) * 8
        for c in range(_NCHUNK):
            for s in range(_CHUNK // _L):
                off = c * _CHUNK + s * _L
                m_v[c, pl.ds(s * _L, _L)] = (
                    qu_v[pl.ds(off, _L)] + kconst)
                m_v[4 + c, pl.ds(s * _L, _L)] = (
                    qv_v[pl.ds(off, _L)] + kconst)
        copies = []
        for c in range(_NCHUNK):
            copies.append(pltpu.async_copy(
                wb_hbm.at[m_v.at[c]],
                gu_v.at[pl.ds(c * _CHUNK, _CHUNK)], sem_u))
            copies.append(pltpu.async_copy(
                hb_hbm.at[m_v.at[4 + c]],
                gv_v.at[pl.ds(c * _CHUNK, _CHUNK)], sem_v))
        for cp in copies:
            cp.wait()
        extract(gu_v, lu_v, ucolsT, k)
        extract(gv_v, lv_v, vcolsT, k)

    def block(blk, carry):
        base = blk * _L
        ucols = [ucolsT[k, pl.ds(base, _L)] for k in range(EMB_K)]
        vcols = [vcolsT[k, pl.ds(base, _L)] for k in range(EMB_K)]
        acc = jnp.zeros((_L,), jnp.float32)
        for j in range(EMB_K):
            w1u = w1_v[j, pl.ds(0, EMB_K)]
            w1v = w1_v[j, pl.ds(EMB_K, EMB_K)]
            h = jnp.full((_L,), b1_vec[j], jnp.float32)
            for k in range(EMB_K):
                h = h + ucols[k] * w1u[k]
            for k in range(EMB_K):
                h = h + vcols[k] * w1v[k]
            h = jnp.maximum(h, 0.0)
            acc = acc + h * w2_vec[j]
        out_v[pl.ds(base, _L)] = acc
        return carry

    lax.fori_loop(0, _NBLK, block, 0)

    pltpu.sync_copy(out_v, out_hbm.at[pl.ds(wid * _BPW, _BPW)])


@functools.cache
def _make_ncf_sc():
  return functools.partial(
    pl.kernel,
    out_type=jax.ShapeDtypeStruct((BATCH,), jnp.float32),
    mesh=plsc.VectorSubcoreMesh(core_axis_name="c", subcore_axis_name="s",
                                num_cores=_NC),
    compiler_params=pltpu.CompilerParams(needs_layout_passes=False,
                                         use_tc_tiling_on_sc=False),
    scratch_types=[
        pltpu.VMEM((_NCHUNK, _CHUNK), jnp.int32),
        pltpu.VMEM((_NCHUNK, _CHUNK), jnp.int32),
        pltpu.VMEM((_BPW,), jnp.int32),
        pltpu.VMEM((_BPW,), jnp.int32),
        pltpu.VMEM((_BPW,), jnp.int32),
        pltpu.VMEM((_BPW,), jnp.int32),
        pltpu.VMEM((2 * _NCHUNK, _CHUNK), jnp.int32),
        pltpu.VMEM((_BPW, EMB_K), jnp.float32),
        pltpu.VMEM((_BPW, EMB_K), jnp.float32),
        pltpu.VMEM((EMB_K, _BPW), jnp.float32),
        pltpu.VMEM((EMB_K, _BPW), jnp.float32),
        pltpu.VMEM((EMB_K, 2 * EMB_K), jnp.float32),
        pltpu.VMEM((EMB_K,), jnp.float32),
        pltpu.VMEM((EMB_K,), jnp.float32),
        pltpu.VMEM((_BPW,), jnp.float32),
        pltpu.SemaphoreType.DMA,
        pltpu.SemaphoreType.DMA,
    ],
  )(_ncf_body)


def kernel(x, W_table, H_table, W1, b1, W2):
    u_idx = x[:, 0].reshape(BATCH // _CHUNK, _CHUNK)
    v_idx = x[:, 1].reshape(BATCH // _CHUNK, _CHUNK)
    wb, hb = _make_detile_tc()(W_table.T, H_table.T)
    out = _make_ncf_sc()(u_idx, v_idx,
                         wb.reshape(_N16, EMB_K), hb.reshape(_N16, EMB_K),
                         W1, b1, W2.reshape(EMB_K))
    return out.reshape(BATCH, 1)


# speedup vs baseline: 10.5617x; 1.0229x over previous
import functools

import jax
import jax.numpy as jnp
from jax import lax
from jax.experimental import pallas as pl
from jax.experimental.pallas import tpu as pltpu
from jax.experimental.pallas import tpu_sc as plsc

BATCH = 16384
EMB_K = 16
NROWS = 1000000

_NC = 2
_NS = 16
_L = 16
_NW = _NC * _NS
_BPW = BATCH // _NW
_CHUNK = 128
_NCHUNK = _BPW // _CHUNK
_NBLK = _BPW // _L

_JB = 1024
_BCOLS = _JB * 128
_NJ = -(-NROWS // _BCOLS)
_BRROWS = _NJ * _JB * 8
_GSTRIDE = _BRROWS * 128 // EMB_K
_N16 = 2 * _GSTRIDE


def _detile_body(wt_ref, ht_ref, wb_ref, hb_ref):
    for src, dst in ((wt_ref, wb_ref), (ht_ref, hb_ref)):
        x = src[...]
        x = x.reshape(8, _JB, 128).transpose(1, 0, 2)
        dst[...] = x.reshape(1, _JB * 8, 128)


@functools.cache
def _make_detile_tc():
    return pl.pallas_call(
        _detile_body,
        grid=(2, _NJ),
        in_specs=[pl.BlockSpec((8, _BCOLS), lambda g, j: (g, j)),
                  pl.BlockSpec((8, _BCOLS), lambda g, j: (g, j))],
        out_specs=[pl.BlockSpec((1, _JB * 8, 128), lambda g, j: (g, j, 0)),
                   pl.BlockSpec((1, _JB * 8, 128), lambda g, j: (g, j, 0))],
        out_shape=[jax.ShapeDtypeStruct((2, _BRROWS, 128), jnp.float32),
                   jax.ShapeDtypeStruct((2, _BRROWS, 128), jnp.float32)],
    )


def _ncf_body(uidx_hbm, vidx_hbm, wb_hbm, hb_hbm, w1_hbm, b1_hbm, w2_hbm,
              out_hbm,
              uidx_v, vidx_v, qu_v, qv_v, lu_v, lv_v, m_v,
              gu_v, gv_v, ucolsT, vcolsT,
              w1_v, b1_v, w2_v, out_v,
              sem_u, sem_v):
    wid = lax.axis_index("s") * _NC + lax.axis_index("c")
    crow = wid * _NCHUNK
    pltpu.sync_copy(uidx_hbm.at[pl.ds(crow, _NCHUNK)], uidx_v)
    pltpu.sync_copy(vidx_hbm.at[pl.ds(crow, _NCHUNK)], vidx_v)
    pltpu.sync_copy(w1_hbm, w1_v)
    pltpu.sync_copy(b1_hbm, b1_v)
    pltpu.sync_copy(w2_hbm, w2_v)

    for idx_ref, q_ref, l_ref in ((uidx_v, qu_v, lu_v),
                                  (vidx_v, qv_v, lv_v)):
        for c in range(_NCHUNK):
            for s in range(_CHUNK // _L):
                idx = idx_ref[c, pl.ds(s * _L, _L)]
                q = ((jnp.right_shift(idx, 7) * 64)
                     | (jnp.right_shift(idx, 4) & 7))
                q_ref[pl.ds(c * _CHUNK + s * _L, _L)] = q
                l_ref[pl.ds(c * _CHUNK + s * _L, _L)] = idx & 15

    lane = lax.iota(jnp.int32, _L)
    b1_vec = b1_v[...]
    w2_vec = w2_v[...]

    def extract(gbuf, lanes, colsT, k):
        def body(b, carry):
            row_ids = b * _L + lane
            lvec = lanes[pl.ds(b * _L, _L)]
            val = plsc.load_gather(gbuf, [row_ids, lvec])
            colsT[k, pl.ds(b * _L, _L)] = val
            return carry
        lax.fori_loop(0, _NBLK, body, 0)

    for k in range(EMB_K):
        kconst = (k // 8) * _GSTRIDE + (k % 8) * 8
        for c in range(_NCHUNK):
            for s in range(_CHUNK // _L):
                off = c * _CHUNK + s * _L
                m_v[c, pl.ds(s * _L, _L)] = (
                    qu_v[pl.ds(off, _L)] + kconst)
                m_v[4 + c, pl.ds(s * _L, _L)] = (
                    qv_v[pl.ds(off, _L)] + kconst)
        copies = []
        for c in range(_NCHUNK):
            copies.append(pltpu.async_copy(
                wb_hbm.at[m_v.at[c]],
                gu_v.at[pl.ds(c * _CHUNK, _CHUNK)], sem_u))
            copies.append(pltpu.async_copy(
                hb_hbm.at[m_v.at[4 + c]],
                gv_v.at[pl.ds(c * _CHUNK, _CHUNK)], sem_v))
        for cp in copies:
            cp.wait()
        extract(gu_v, lu_v, ucolsT, k)
        extract(gv_v, lv_v, vcolsT, k)

    def block(blk, carry):
        base = blk * _L
        ucols = [ucolsT[k, pl.ds(base, _L)] for k in range(EMB_K)]
        vcols = [vcolsT[k, pl.ds(base, _L)] for k in range(EMB_K)]
        acc = jnp.zeros((_L,), jnp.float32)
        for j in range(EMB_K):
            w1u = w1_v[j, pl.ds(0, EMB_K)]
            w1v = w1_v[j, pl.ds(EMB_K, EMB_K)]
            h = jnp.full((_L,), b1_vec[j], jnp.float32)
            for k in range(EMB_K):
                h = h + ucols[k] * w1u[k]
            for k in range(EMB_K):
                h = h + vcols[k] * w1v[k]
            h = jnp.maximum(h, 0.0)
            acc = acc + h * w2_vec[j]
        out_v[pl.ds(base, _L)] = acc
        return carry

    lax.fori_loop(0, _NBLK, block, 0)

    pltpu.sync_copy(out_v, out_hbm.at[pl.ds(wid * _BPW, _BPW)])


@functools.cache
def _make_ncf_sc():
  return functools.partial(
    pl.kernel,
    out_type=jax.ShapeDtypeStruct((BATCH,), jnp.float32),
    mesh=plsc.VectorSubcoreMesh(core_axis_name="c", subcore_axis_name="s",
                                num_cores=_NC),
    compiler_params=pltpu.CompilerParams(needs_layout_passes=False,
                                         use_tc_tiling_on_sc=False),
    scratch_types=[
        pltpu.VMEM((_NCHUNK, _CHUNK), jnp.int32),
        pltpu.VMEM((_NCHUNK, _CHUNK), jnp.int32),
        pltpu.VMEM((_BPW,), jnp.int32),
        pltpu.VMEM((_BPW,), jnp.int32),
        pltpu.VMEM((_BPW,), jnp.int32),
        pltpu.VMEM((_BPW,), jnp.int32),
        pltpu.VMEM((2 * _NCHUNK, _CHUNK), jnp.int32),
        pltpu.VMEM((_BPW, EMB_K), jnp.float32),
        pltpu.VMEM((_BPW, EMB_K), jnp.float32),
        pltpu.VMEM((EMB_K, _BPW), jnp.float32),
        pltpu.VMEM((EMB_K, _BPW), jnp.float32),
        pltpu.VMEM((EMB_K, 2 * EMB_K), jnp.float32),
        pltpu.VMEM((EMB_K,), jnp.float32),
        pltpu.VMEM((EMB_K,), jnp.float32),
        pltpu.VMEM((_BPW,), jnp.float32),
        pltpu.SemaphoreType.DMA,
        pltpu.SemaphoreType.DMA,
    ],
  )(_ncf_body)


def kernel(x, W_table, H_table, W1, b1, W2):
    u_idx = x[:, 0].reshape(BATCH // _CHUNK, _CHUNK)
    v_idx = x[:, 1].reshape(BATCH // _CHUNK, _CHUNK)
    wb, hb = _make_detile_tc()(W_table.T, H_table.T)
    out = _make_ncf_sc()(u_idx, v_idx,
                         wb.reshape(_N16, EMB_K), hb.reshape(_N16, EMB_K),
                         W1, b1, W2.reshape(EMB_K))
    return out.reshape(BATCH, 1)


# speedup vs baseline: 11.4071x; 1.0800x over previous
import functools

import jax
import jax.numpy as jnp
from jax import lax
from jax.experimental import pallas as pl
from jax.experimental.pallas import tpu as pltpu
from jax.experimental.pallas import tpu_sc as plsc

BATCH = 16384
EMB_K = 16
NROWS = 1000000

_NC = 2
_NS = 16
_L = 16
_NW = _NC * _NS
_BPW = BATCH // _NW
_CHUNK = 128
_NCHUNK = _BPW // _CHUNK
_NBLK = _BPW // _L

_JB = 1024
_BCOLS = _JB * 128
_NJ = -(-NROWS // _BCOLS)
_BRROWS = _NJ * _JB * 8
_GSTRIDE = _BRROWS * 128 // EMB_K
_N16 = 2 * _GSTRIDE


def _detile_body(wt_ref, ht_ref, wb_ref, hb_ref):
    for src, dst in ((wt_ref, wb_ref), (ht_ref, hb_ref)):
        x = src[...]
        x = x.reshape(8, _JB, 128).transpose(1, 0, 2)
        dst[...] = x.reshape(1, _JB * 8, 128)


@functools.cache
def _make_detile_tc():
    return pl.pallas_call(
        _detile_body,
        grid=(2, _NJ),
        in_specs=[pl.BlockSpec((8, _BCOLS), lambda g, j: (g, j)),
                  pl.BlockSpec((8, _BCOLS), lambda g, j: (g, j))],
        out_specs=[pl.BlockSpec((1, _JB * 8, 128), lambda g, j: (g, j, 0)),
                   pl.BlockSpec((1, _JB * 8, 128), lambda g, j: (g, j, 0))],
        out_shape=[jax.ShapeDtypeStruct((2, _BRROWS, 128), jnp.float32),
                   jax.ShapeDtypeStruct((2, _BRROWS, 128), jnp.float32)],
    )


def _ncf_body(uidx_hbm, vidx_hbm, wb_hbm, hb_hbm, w1_hbm, b1_hbm, w2_hbm,
              out_hbm,
              uidx_v, vidx_v, qu_v, qv_v, lu_v, lv_v, m_v,
              gu_v, gv_v, ucolsT, vcolsT,
              w1_v, b1_v, w2_v, out_v,
              sem_u, sem_v):
    wid = lax.axis_index("s") * _NC + lax.axis_index("c")
    crow = wid * _NCHUNK
    pltpu.sync_copy(uidx_hbm.at[pl.ds(crow, _NCHUNK)], uidx_v)
    pltpu.sync_copy(vidx_hbm.at[pl.ds(crow, _NCHUNK)], vidx_v)
    pltpu.sync_copy(w1_hbm, w1_v)
    pltpu.sync_copy(b1_hbm, b1_v)
    pltpu.sync_copy(w2_hbm, w2_v)

    for idx_ref, q_ref, l_ref in ((uidx_v, qu_v, lu_v),
                                  (vidx_v, qv_v, lv_v)):
        for c in range(_NCHUNK):
            for s in range(_CHUNK // _L):
                idx = idx_ref[c, pl.ds(s * _L, _L)]
                q = ((jnp.right_shift(idx, 7) * 64)
                     | (jnp.right_shift(idx, 4) & 7))
                q_ref[pl.ds(c * _CHUNK + s * _L, _L)] = q
                l_ref[pl.ds(c * _CHUNK + s * _L, _L)] = idx & 15

    lane = lax.iota(jnp.int32, _L)
    b1_vec = b1_v[...]
    w2_vec = w2_v[...]

    def extract(gbuf, lanes, colsT, k):
        def body(b, carry):
            row_ids = b * _L + lane
            lvec = lanes[pl.ds(b * _L, _L)]
            val = plsc.load_gather(gbuf, [row_ids, lvec])
            colsT[k, pl.ds(b * _L, _L)] = val
            return carry
        lax.fori_loop(0, _NBLK, body, 0)

    sems = (sem_u, sem_v)
    inflight = [None, None]

    def fire(k):
        p = k % 2
        kconst = (k // 8) * _GSTRIDE + (k % 8) * 8
        for c in range(_NCHUNK):
            for s in range(_CHUNK // _L):
                off = c * _CHUNK + s * _L
                m_v[p, c, pl.ds(s * _L, _L)] = (
                    qu_v[pl.ds(off, _L)] + kconst)
                m_v[p, 4 + c, pl.ds(s * _L, _L)] = (
                    qv_v[pl.ds(off, _L)] + kconst)
        copies = []
        for c in range(_NCHUNK):
            copies.append(pltpu.async_copy(
                wb_hbm.at[m_v.at[p, c]],
                gu_v.at[p, pl.ds(c * _CHUNK, _CHUNK)], sems[p]))
            copies.append(pltpu.async_copy(
                hb_hbm.at[m_v.at[p, 4 + c]],
                gv_v.at[p, pl.ds(c * _CHUNK, _CHUNK)], sems[p]))
        inflight[p] = copies

    fire(0)
    for k in range(EMB_K):
        if k + 1 < EMB_K:
            fire(k + 1)
        p = k % 2
        for cp in inflight[p]:
            cp.wait()
        extract(gu_v.at[p], lu_v, ucolsT, k)
        extract(gv_v.at[p], lv_v, vcolsT, k)

    def block(blk, carry):
        base = blk * _L
        ucols = [ucolsT[k, pl.ds(base, _L)] for k in range(EMB_K)]
        vcols = [vcolsT[k, pl.ds(base, _L)] for k in range(EMB_K)]
        acc = jnp.zeros((_L,), jnp.float32)
        for j in range(EMB_K):
            w1u = w1_v[j, pl.ds(0, EMB_K)]
            w1v = w1_v[j, pl.ds(EMB_K, EMB_K)]
            h = jnp.full((_L,), b1_vec[j], jnp.float32)
            for k in range(EMB_K):
                h = h + ucols[k] * w1u[k]
            for k in range(EMB_K):
                h = h + vcols[k] * w1v[k]
            h = jnp.maximum(h, 0.0)
            acc = acc + h * w2_vec[j]
        out_v[pl.ds(base, _L)] = acc
        return carry

    lax.fori_loop(0, _NBLK, block, 0)

    pltpu.sync_copy(out_v, out_hbm.at[pl.ds(wid * _BPW, _BPW)])


@functools.cache
def _make_ncf_sc():
  return functools.partial(
    pl.kernel,
    out_type=jax.ShapeDtypeStruct((BATCH,), jnp.float32),
    mesh=plsc.VectorSubcoreMesh(core_axis_name="c", subcore_axis_name="s",
                                num_cores=_NC),
    compiler_params=pltpu.CompilerParams(needs_layout_passes=False,
                                         use_tc_tiling_on_sc=False),
    scratch_types=[
        pltpu.VMEM((_NCHUNK, _CHUNK), jnp.int32),
        pltpu.VMEM((_NCHUNK, _CHUNK), jnp.int32),
        pltpu.VMEM((_BPW,), jnp.int32),
        pltpu.VMEM((_BPW,), jnp.int32),
        pltpu.VMEM((_BPW,), jnp.int32),
        pltpu.VMEM((_BPW,), jnp.int32),
        pltpu.VMEM((2, 2 * _NCHUNK, _CHUNK), jnp.int32),
        pltpu.VMEM((2, _BPW, EMB_K), jnp.float32),
        pltpu.VMEM((2, _BPW, EMB_K), jnp.float32),
        pltpu.VMEM((EMB_K, _BPW), jnp.float32),
        pltpu.VMEM((EMB_K, _BPW), jnp.float32),
        pltpu.VMEM((EMB_K, 2 * EMB_K), jnp.float32),
        pltpu.VMEM((EMB_K,), jnp.float32),
        pltpu.VMEM((EMB_K,), jnp.float32),
        pltpu.VMEM((_BPW,), jnp.float32),
        pltpu.SemaphoreType.DMA,
        pltpu.SemaphoreType.DMA,
    ],
  )(_ncf_body)


def kernel(x, W_table, H_table, W1, b1, W2):
    u_idx = x[:, 0].reshape(BATCH // _CHUNK, _CHUNK)
    v_idx = x[:, 1].reshape(BATCH // _CHUNK, _CHUNK)
    wb, hb = _make_detile_tc()(W_table.T, H_table.T)
    out = _make_ncf_sc()(u_idx, v_idx,
                         wb.reshape(_N16, EMB_K), hb.reshape(_N16, EMB_K),
                         W1, b1, W2.reshape(EMB_K))
    return out.reshape(BATCH, 1)
